# baseline (speedup 1.0000x reference)
import functools

import jax
import jax.numpy as jnp
from jax import lax
from jax.experimental import pallas as pl
from jax.experimental.pallas import tpu as pltpu
from jax.experimental.pallas import tpu_sc as plsc

VOCAB = 1000000
EMBED_DIM = 32
BATCH = 4096
HIST = 200

NUM_CORES = 2
NUM_SUBCORES = 16
NW = NUM_CORES * NUM_SUBCORES
NBUF = 4

_mesh = plsc.VectorSubcoreMesh(core_axis_name="c", subcore_axis_name="s")


@functools.partial(
    pl.kernel,
    mesh=_mesh,
    out_type=jax.ShapeDtypeStruct((HIST, 4, NW, 8, 128), jnp.float32),
    scratch_types=[
        pltpu.VMEM((HIST // 8, 8, 128), jnp.int32),
        [pltpu.VMEM((128, EMBED_DIM), jnp.float32) for _ in range(NBUF)],
        [pltpu.VMEM((4, 8, 128), jnp.float32) for _ in range(NBUF)],
        [pltpu.SemaphoreType.DMA for _ in range(NBUF)],
        [pltpu.SemaphoreType.DMA for _ in range(NBUF)],
    ],
    compiler_params=pltpu.CompilerParams(
        use_tc_tiling_on_sc=False, needs_layout_passes=False
    ),
)
def _gather_kernel(xp_hbm, table_hbm, out_hbm, idx_all, rows, outblk,
                   gsems, wsems):
    bt = lax.axis_index("s") * NUM_CORES + lax.axis_index("c")

    pltpu.sync_copy(xp_hbm.at[:, bt], idx_all)

    def issue(u, b):
        idx_list = idx_all.at[u // 8, u % 8]
        pltpu.async_copy(table_hbm.at[idx_list], rows[b], gsems[b])

    for j in range(NBUF - 1):
        issue(j, j)

    rowbase = [lax.iota(jnp.int32, 16) + blk * 16 for blk in range(8)]

    def quad(p, carry):
        for j in range(NBUF):
            u = p * NBUF + j
            pltpu.make_async_copy(
                table_hbm.at[idx_all.at[u // 8, u % 8]],
                rows[j], gsems[j]).wait()

            @pl.when(u + NBUF - 1 < HIST)
            def _():
                issue(u + NBUF - 1, (j + NBUF - 1) % NBUF)

            @pl.when(u >= NBUF)
            def _():
                pltpu.make_async_copy(
                    outblk[j], out_hbm.at[u - NBUF, :, bt], wsems[j]).wait()

            coli = jnp.zeros((16,), jnp.int32)
            for d in range(EMBED_DIM):
                for blk in range(8):
                    vals = plsc.load_gather(rows[j], [rowbase[blk], coli])
                    outblk[j][d // 8, d % 8, pl.ds(blk * 16, 16)] = vals
                coli = coli + 1

            pltpu.async_copy(outblk[j], out_hbm.at[u, :, bt], wsems[j])
        return carry

    lax.fori_loop(0, HIST // NBUF, quad, 0)

    for j in range(NBUF):
        pltpu.make_async_copy(
            outblk[j], out_hbm.at[HIST - NBUF + j, :, bt], wsems[j]).wait()


def kernel(x, vectors):
    xp = x.T.reshape(HIST // 8, 8, NW, 128).transpose(0, 2, 1, 3)
    out5 = _gather_kernel(xp, vectors)
    return out5.transpose(2, 4, 0, 1, 3).reshape(BATCH, HIST, EMBED_DIM)


# speedup vs baseline: 1.1698x; 1.1698x over previous
import functools

import jax
import jax.numpy as jnp
from jax import lax
from jax.experimental import pallas as pl
from jax.experimental.pallas import tpu as pltpu
from jax.experimental.pallas import tpu_sc as plsc

VOCAB = 1000000
EMBED_DIM = 32
BATCH = 4096
HIST = 200

NUM_CORES = 2
NUM_SUBCORES = 16
NW = NUM_CORES * NUM_SUBCORES
NBUF = 4
BLK = 4 * 8 * 128

_mesh = plsc.VectorSubcoreMesh(core_axis_name="c", subcore_axis_name="s")


@functools.partial(
    pl.kernel,
    mesh=_mesh,
    out_type=jax.ShapeDtypeStruct((HIST * 4 * NW * 8 * 128,), jnp.float32),
    scratch_types=[
        pltpu.VMEM((HIST // 8, 8, 128), jnp.int32),
        [pltpu.VMEM((128, EMBED_DIM), jnp.float32) for _ in range(NBUF)],
        [pltpu.VMEM((BLK,), jnp.float32) for _ in range(NBUF)],
        [pltpu.SemaphoreType.DMA for _ in range(NBUF)],
        [pltpu.SemaphoreType.DMA for _ in range(NBUF)],
    ],
    compiler_params=pltpu.CompilerParams(
        use_tc_tiling_on_sc=False, needs_layout_passes=False
    ),
)
def _gather_kernel(xp_hbm, table_hbm, out_hbm, idx_all, rows, outblk,
                   gsems, wsems):
    bt = lax.axis_index("s") * NUM_CORES + lax.axis_index("c")

    pltpu.sync_copy(xp_hbm.at[:, bt], idx_all)

    def issue(u, b):
        idx_list = idx_all.at[u // 8, u % 8]
        pltpu.async_copy(table_hbm.at[idx_list], rows[b], gsems[b])

    for j in range(NBUF - 1):
        issue(j, j)

    def wb_copy(u, b, dt):
        off = ((u * 4 + dt) * NW + bt) * 1024
        return pltpu.make_async_copy(
            outblk[b].at[pl.ds(dt * 1024, 1024)],
            out_hbm.at[pl.ds(off, 1024)], wsems[b])

    def quad(p, carry):
        for j in range(NBUF):
            u = p * NBUF + j
            pltpu.make_async_copy(
                table_hbm.at[idx_all.at[u // 8, u % 8]],
                rows[j], gsems[j]).wait()

            @pl.when(u + NBUF - 1 < HIST)
            def _():
                issue(u + NBUF - 1, (j + NBUF - 1) % NBUF)

            @pl.when(u >= NBUF)
            def _():
                for dt in range(4):
                    wb_copy(u - NBUF, j, dt).wait()

            i16 = lax.iota(jnp.int32, 16)
            sbase = [((i16 + 16 * h) // 8) * 1024
                     + ((i16 + 16 * h) % 8) * 128 for h in range(2)]
            for h in range(2):
                sb = sbase[h]
                for bi in range(128):
                    vals = rows[j][bi, pl.ds(16 * h, 16)]
                    plsc.store_scatter(outblk[j], [sb + bi], vals)

            for dt in range(4):
                wb_copy(u, j, dt).start()
        return carry

    lax.fori_loop(0, HIST // NBUF, quad, 0)

    for j in range(NBUF):
        for dt in range(4):
            wb_copy(HIST - NBUF + j, j, dt).wait()


def kernel(x, vectors):
    xp = x.T.reshape(HIST // 8, 8, NW, 128).transpose(0, 2, 1, 3)
    outf = _gather_kernel(xp, vectors)
    out5 = outf.reshape(HIST, 4, NW, 8, 128)
    return out5.transpose(2, 4, 0, 1, 3).reshape(BATCH, HIST, EMBED_DIM)


# speedup vs baseline: 1.3153x; 1.1244x over previous
import functools

import jax
import jax.numpy as jnp
from jax import lax
from jax.experimental import pallas as pl
from jax.experimental.pallas import tpu as pltpu
from jax.experimental.pallas import tpu_sc as plsc

VOCAB = 1000000
EMBED_DIM = 32
BATCH = 4096
HIST = 200

NUM_CORES = 2
NUM_SUBCORES = 16
NW = NUM_CORES * NUM_SUBCORES
NBUF = 4
BLK = 4 * 8 * 128

_mesh = plsc.VectorSubcoreMesh(core_axis_name="c", subcore_axis_name="s")


@functools.partial(
    pl.kernel,
    mesh=_mesh,
    out_type=jax.ShapeDtypeStruct((HIST * 4 * NW * 8 * 128,), jnp.float32),
    scratch_types=[
        pltpu.VMEM((HIST // 8, 8, 128), jnp.int32),
        [pltpu.VMEM((128, EMBED_DIM), jnp.float32) for _ in range(NBUF)],
        [pltpu.VMEM((BLK,), jnp.float32) for _ in range(NBUF)],
        [pltpu.SemaphoreType.DMA for _ in range(NBUF)],
        [pltpu.SemaphoreType.DMA for _ in range(NBUF)],
    ],
    compiler_params=pltpu.CompilerParams(
        use_tc_tiling_on_sc=False, needs_layout_passes=False
    ),
)
def _gather_kernel(xp_hbm, table_hbm, out_hbm, idx_all, rows, outblk,
                   gsems, wsems):
    bt = lax.axis_index("s") * NUM_CORES + lax.axis_index("c")

    pltpu.sync_copy(xp_hbm.at[:, bt], idx_all)

    def issue(u, b):
        idx_list = idx_all.at[u // 8, u % 8]
        pltpu.async_copy(table_hbm.at[idx_list], rows[b], gsems[b])

    for j in range(NBUF - 1):
        issue(j, j)

    def wb_copy(u, b, dt):
        off = ((u * 4 + dt) * NW + bt) * 1024
        return pltpu.make_async_copy(
            outblk[b].at[pl.ds(dt * 1024, 1024)],
            out_hbm.at[pl.ds(off, 1024)], wsems[b])

    def quad(p, carry):
        for j in range(NBUF):
            u = p * NBUF + j
            pltpu.make_async_copy(
                table_hbm.at[idx_all.at[u // 8, u % 8]],
                rows[j], gsems[j]).wait()

            @pl.when(u + NBUF - 1 < HIST)
            def _():
                issue(u + NBUF - 1, (j + NBUF - 1) % NBUF)

            @pl.when(u >= NBUF)
            def _():
                for dt in range(4):
                    wb_copy(u - NBUF, j, dt).wait()

            i16 = lax.iota(jnp.int32, 16)
            sbase = [((i16 + 16 * h) // 8) * 1024
                     + ((i16 + 16 * h) % 8) * 128 for h in range(2)]
            for h in range(2):
                sb = sbase[h]
                for g in range(0, 128, 8):
                    vals = [rows[j][bi, pl.ds(16 * h, 16)]
                            for bi in range(g, g + 8)]
                    for k, bi in enumerate(range(g, g + 8)):
                        plsc.store_scatter(outblk[j], [sb + bi], vals[k])

            for dt in range(4):
                wb_copy(u, j, dt).start()
        return carry

    lax.fori_loop(0, HIST // NBUF, quad, 0)

    for j in range(NBUF):
        for dt in range(4):
            wb_copy(HIST - NBUF + j, j, dt).wait()


def kernel(x, vectors):
    xp = x.T.reshape(HIST // 8, 8, NW, 128).transpose(0, 2, 1, 3)
    outf = _gather_kernel(xp, vectors)
    out5 = outf.reshape(HIST, 4, NW, 8, 128)
    return out5.transpose(2, 4, 0, 1, 3).reshape(BATCH, HIST, EMBED_DIM)
